# Initial kernel scaffold; baseline (speedup 1.0000x reference)
import jax
import jax.numpy as jnp
from jax.experimental import pallas as pl


def kernel(ids, table):
    raise NotImplementedError("write your pallas kernel here")


# speedup vs baseline: 1.6836x; 1.6836x over previous
import functools

import jax
import jax.numpy as jnp
from jax import lax
from jax.experimental import pallas as pl
from jax.experimental.pallas import tpu as pltpu
from jax.experimental.pallas import tpu_sc as plsc

_NC = 2
_NS = 16
_NW = _NC * _NS

_CHUNK = 128


@functools.partial(jax.jit, static_argnames=())
def _embed(ids_flat, table):
    (B,) = ids_flat.shape
    V, D = table.shape
    b_per_w = B // _NW
    n_chunks = b_per_w // _CHUNK

    mesh = plsc.VectorSubcoreMesh(core_axis_name="c", subcore_axis_name="s")

    @functools.partial(
        pl.kernel,
        out_type=jax.ShapeDtypeStruct((B, D), jnp.float32),
        mesh=mesh,
        scratch_types=[
            pltpu.VMEM((b_per_w,), jnp.int32),
            pltpu.VMEM((_CHUNK, D), jnp.float32),
            pltpu.SemaphoreType.DMA,
        ],
        compiler_params=pltpu.CompilerParams(use_tc_tiling_on_sc=False),
    )
    def body(table_hbm, ids_hbm, out_hbm, idx_v, rows_v, sem):
        wid = lax.axis_index("s") * _NC + lax.axis_index("c")
        base = wid * b_per_w
        pltpu.sync_copy(ids_hbm.at[pl.ds(base, b_per_w)], idx_v)

        @pl.loop(0, n_chunks)
        def _(i):
            off = i * _CHUNK
            pltpu.async_copy(
                table_hbm.at[idx_v.at[pl.ds(off, _CHUNK)]], rows_v, sem
            ).wait()
            pltpu.sync_copy(rows_v, out_hbm.at[pl.ds(base + off, _CHUNK)])

    return body(table, ids_flat)


def kernel(ids, table):
    S0, S1 = ids.shape
    D = table.shape[1]
    ids_flat = ids.reshape(S0 * S1).astype(jnp.int32)
    out = _embed(ids_flat, table)
    return out.reshape(S0, S1, D)


# speedup vs baseline: 1.8779x; 1.1154x over previous
import functools

import jax
import jax.numpy as jnp
from jax import lax
from jax.experimental import pallas as pl
from jax.experimental.pallas import tpu as pltpu
from jax.experimental.pallas import tpu_sc as plsc

_NC = 2
_NS = 16
_NW = _NC * _NS

_CHUNK = 128
_NBUF = 4


@functools.partial(jax.jit, static_argnames=())
def _embed(ids_flat, table):
    (B,) = ids_flat.shape
    V, D = table.shape
    b_per_w = B // _NW
    n_chunks = b_per_w // _CHUNK

    mesh = plsc.VectorSubcoreMesh(core_axis_name="c", subcore_axis_name="s")

    @functools.partial(
        pl.kernel,
        out_type=jax.ShapeDtypeStruct((B, D), jnp.float32),
        mesh=mesh,
        scratch_types=[
            pltpu.VMEM((b_per_w,), jnp.int32),
            *[pltpu.VMEM((_CHUNK, D), jnp.float32) for _ in range(_NBUF)],
            *[pltpu.SemaphoreType.DMA for _ in range(2 * _NBUF)],
        ],
        compiler_params=pltpu.CompilerParams(use_tc_tiling_on_sc=False),
    )
    def body(table_hbm, ids_hbm, out_hbm, idx_v, *scratch):
        rows = scratch[:_NBUF]
        gsem = scratch[_NBUF : 2 * _NBUF]
        wsem = scratch[2 * _NBUF :]
        wid = lax.axis_index("s") * _NC + lax.axis_index("c")
        base = wid * b_per_w
        pltpu.sync_copy(ids_hbm.at[pl.ds(base, b_per_w)], idx_v)

        def gather_desc(g, b):
            return pltpu.make_async_copy(
                table_hbm.at[idx_v.at[pl.ds(g * _CHUNK, _CHUNK)]], rows[b], gsem[b]
            )

        def write_desc(g, b):
            return pltpu.make_async_copy(
                rows[b], out_hbm.at[pl.ds(base + g * _CHUNK, _CHUNK)], wsem[b]
            )

        @pl.loop(0, n_chunks + _NBUF, step=_NBUF)
        def _(i):
            for b in range(_NBUF):
                g = i + b

                @pl.when(g >= _NBUF)
                def _():
                    write_desc(g - _NBUF, b).wait()

                @pl.when(g < n_chunks)
                def _():
                    gather_desc(g, b).start()

                h = g - (_NBUF - 1)
                bh = (b + 1) % _NBUF

                @pl.when(jnp.logical_and(h >= 0, h < n_chunks))
                def _():
                    gather_desc(h, bh).wait()
                    write_desc(h, bh).start()

    return body(table, ids_flat)


def kernel(ids, table):
    S0, S1 = ids.shape
    D = table.shape[1]
    ids_flat = ids.reshape(S0 * S1).astype(jnp.int32)
    out = _embed(ids_flat, table)
    return out.reshape(S0, S1, D)
